# Initial kernel scaffold; baseline (speedup 1.0000x reference)
import jax
import jax.numpy as jnp
from jax.experimental import pallas as pl


def kernel(x, segment_ids):
    raise NotImplementedError("write your pallas kernel here")


# speedup vs baseline: 2.5278x; 2.5278x over previous
import functools

import jax
import jax.numpy as jnp
from jax import lax
from jax.experimental import pallas as pl
from jax.experimental.pallas import tpu as pltpu
from jax.experimental.pallas import tpu_sc as plsc

N_ROWS = 100000
N_FEAT = 128
N_SEG = 512

N_CORES = 2
N_SUBCORES = 16
ROWS_PER_W = N_ROWS // N_SUBCORES
COLS_PER_C = N_FEAT // N_CORES
CHUNK = 640
FULL_CHUNKS = ROWS_PER_W // CHUNK
LAST_CHUNK = ROWS_PER_W - FULL_CHUNKS * CHUNK
LAST_GROUPS = LAST_CHUNK // 16
LAST_TAIL = LAST_CHUNK - LAST_GROUPS * 16
IDS_BUF = ROWS_PER_W + 6


def _body(x_hbm, ids_hbm, out_hbm, ids_v, buf, acc, idx2, acc_sh):
    c = lax.axis_index("c")
    s = lax.axis_index("s")
    row0 = s * ROWS_PER_W
    col0 = c * COLS_PER_C

    zeros16 = jnp.zeros((16,), jnp.float32)

    def zero_row(r, _):
        for p in range(COLS_PER_C // 16):
            acc[r, pl.ds(p * 16, 16)] = zeros16
        return 0

    lax.fori_loop(0, N_SEG, zero_row, 0)

    pltpu.sync_copy(acc.at[pl.ds(0, 32)], acc_sh.at[pl.ds(s * 32, 32)])

    iota16 = lax.iota(jnp.int32, 16)
    for i in range(16):
        idx2[i, pl.ds(0, 16)] = iota16 + (32 * i)
        idx2[i, pl.ds(16, 16)] = iota16 + (32 * i + 16)

    plsc.subcore_barrier()

    start_al = (row0 // 8) * 8
    d = row0 - start_al
    pltpu.sync_copy(ids_hbm.at[pl.ds(start_al, IDS_BUF)], ids_v)

    def add_row(local_row, seg):
        for p in range(COLS_PER_C // 16):
            v = buf[local_row, pl.ds(p * 16, 16)]
            plsc.addupdate(acc.at[seg, pl.ds(p * 16, 16)], v)

    def group_body(chunk_base, g):
        ids16 = ids_v[pl.ds(d + chunk_base + g * 16, 16)]
        for j in range(16):
            add_row(g * 16 + j, ids16[j])

    for k in range(FULL_CHUNKS + 1):
        rows_k = CHUNK if k < FULL_CHUNKS else LAST_CHUNK
        groups_k = rows_k // 16
        pltpu.sync_copy(
            x_hbm.at[pl.ds(row0 + k * CHUNK, rows_k),
                     pl.ds(col0, COLS_PER_C)],
            buf.at[pl.ds(0, rows_k)],
        )
        chunk_base = k * CHUNK

        def loop_body(g, _, chunk_base=chunk_base):
            group_body(chunk_base, g)
            return 0

        lax.fori_loop(0, groups_k, loop_body, 0)

        if k == FULL_CHUNKS and LAST_TAIL:
            ids16 = ids_v[pl.ds(d + ROWS_PER_W - 16, 16)]
            for j in range(16 - LAST_TAIL, 16):
                add_row(groups_k * 16 + j - (16 - LAST_TAIL), ids16[j])

    for i in range(16):
        pltpu.sync_copy(acc.at[pl.ds(32 * i, 32)],
                        acc_sh.at[idx2.at[i]], add=True)

    plsc.subcore_barrier()

    pltpu.sync_copy(
        acc_sh.at[pl.ds(s * 32, 32)],
        out_hbm.at[pl.ds(s * 32, 32), pl.ds(col0, COLS_PER_C)],
    )


@jax.jit
def kernel(x, segment_ids):
    ids32 = segment_ids.astype(jnp.int32)
    mesh = plsc.VectorSubcoreMesh(
        core_axis_name="c", subcore_axis_name="s",
        num_cores=N_CORES, num_subcores=N_SUBCORES)
    f = pl.kernel(
        _body,
        out_type=jax.ShapeDtypeStruct((N_SEG, N_FEAT), jnp.float32),
        mesh=mesh,
        compiler_params=pltpu.CompilerParams(use_tc_tiling_on_sc=False),
        scratch_types=[
            pltpu.VMEM((IDS_BUF,), jnp.int32),
            pltpu.VMEM((CHUNK, COLS_PER_C), jnp.float32),
            pltpu.VMEM((N_SEG, COLS_PER_C), jnp.float32),
            pltpu.VMEM((16, 32), jnp.int32),
            pltpu.VMEM_SHARED((N_SEG, COLS_PER_C), jnp.float32),
        ],
    )
    return f(x, ids32)


# speedup vs baseline: 4.9982x; 1.9773x over previous
import functools

import jax
import jax.numpy as jnp
from jax import lax
from jax.experimental import pallas as pl
from jax.experimental.pallas import tpu as pltpu
from jax.experimental.pallas import tpu_sc as plsc

N_ROWS = 100000
N_FEAT = 128
N_SEG = 512

N_CORES = 2
N_SUBCORES = 16
ROWS_PER_W = N_ROWS // N_SUBCORES
COLS_PER_C = N_FEAT // N_CORES
NP16 = COLS_PER_C // 16
CHUNK = 640
FULL_CHUNKS = ROWS_PER_W // CHUNK
LAST_CHUNK = ROWS_PER_W - FULL_CHUNKS * CHUNK
LAST_GROUPS = LAST_CHUNK // 16
LAST_TAIL = LAST_CHUNK - LAST_GROUPS * 16
N_CHUNKS = FULL_CHUNKS + 1
IDS_BUF = ROWS_PER_W + 6


def _body(x_hbm, ids_hbm, out_hbm, ids_v, buf0, buf1, acc, zbuf, idx2, sem0,
          sem1, acc_sh):
    c = lax.axis_index("c")
    s = lax.axis_index("s")
    row0 = s * ROWS_PER_W
    col0 = c * COLS_PER_C

    zeros16 = jnp.zeros((16,), jnp.float32)
    iota16 = lax.iota(jnp.int32, 16)

    for r in range(32):
        for p in range(NP16):
            zbuf[r, pl.ds(p * 16, 16)] = zeros16

    pltpu.sync_copy(zbuf, acc_sh.at[pl.ds(s * 32, 32)])

    for i in range(16):
        idx2[i, pl.ds(0, 16)] = iota16 + (32 * i)
        idx2[i, pl.ds(16, 16)] = iota16 + (32 * i + 16)

    start_al = (row0 // 8) * 8
    d = row0 - start_al
    pltpu.sync_copy(ids_hbm.at[pl.ds(start_al, IDS_BUF)], ids_v)

    first_id = ids_v[pl.ds(d, 16)][0]
    last_id = ids_v[pl.ds(d + ROWS_PER_W - 16, 16)][15]
    blk_lo = first_id // 32
    blk_hi = last_id // 32

    def zero_row(r, _):
        for p in range(NP16):
            acc[r, pl.ds(p * 16, 16)] = zeros16
        return 0

    lax.fori_loop(blk_lo * 32, blk_hi * 32 + 32, zero_row, 0)

    plsc.subcore_barrier()

    bufs = [buf0, buf1]
    sems = [sem0, sem1]

    def start_dma(k):
        rows_k = CHUNK if k < FULL_CHUNKS else LAST_CHUNK
        return pltpu.async_copy(
            x_hbm.at[pl.ds(row0 + k * CHUNK, rows_k), pl.ds(col0, COLS_PER_C)],
            bufs[k % 2].at[pl.ds(0, rows_k)],
            sems[k % 2],
        )

    def add_row(buf, local_row, seg):
        for p in range(NP16):
            v = buf[local_row, pl.ds(p * 16, 16)]
            plsc.addupdate(acc.at[seg, pl.ds(p * 16, 16)], v)

    def group_body(buf, chunk_base, g):
        base = g * 16
        ids16 = ids_v[pl.ds(d + chunk_base + base, 16)]
        seg0 = ids16[0]
        seg15 = ids16[15]

        def fast(_):
            for p in range(NP16):
                vs = [buf[base + j, pl.ds(p * 16, 16)] for j in range(16)]
                while len(vs) > 1:
                    vs = [vs[i] + vs[i + 1] for i in range(0, len(vs), 2)]
                plsc.addupdate(acc.at[seg0, pl.ds(p * 16, 16)], vs[0])
            return 0

        def slow(_):
            for j in range(16):
                add_row(buf, base + j, ids16[j])
            return 0

        lax.cond(seg0 == seg15, fast, slow, 0)

    descs = [None, None]
    descs[0] = start_dma(0)
    for k in range(N_CHUNKS):
        if k + 1 < N_CHUNKS:
            descs[(k + 1) % 2] = start_dma(k + 1)
        descs[k % 2].wait()
        buf = bufs[k % 2]
        chunk_base = k * CHUNK
        groups_k = (CHUNK if k < FULL_CHUNKS else LAST_CHUNK) // 16

        def loop_body(g, _, buf=buf, chunk_base=chunk_base):
            group_body(buf, chunk_base, g)
            return 0

        lax.fori_loop(0, groups_k, loop_body, 0)

        if k == FULL_CHUNKS and LAST_TAIL:
            ids16 = ids_v[pl.ds(d + ROWS_PER_W - 16, 16)]
            for j in range(16 - LAST_TAIL, 16):
                add_row(buf, groups_k * 16 + j - (16 - LAST_TAIL), ids16[j])

    def combine(i, _):
        pltpu.sync_copy(acc.at[pl.ds(32 * i, 32)],
                        acc_sh.at[idx2.at[i]], add=True)
        return 0

    lax.fori_loop(blk_lo, blk_hi + 1, combine, 0)

    plsc.subcore_barrier()

    pltpu.sync_copy(
        acc_sh.at[pl.ds(s * 32, 32)],
        out_hbm.at[pl.ds(s * 32, 32), pl.ds(col0, COLS_PER_C)],
    )


@jax.jit
def kernel(x, segment_ids):
    ids32 = segment_ids.astype(jnp.int32)
    mesh = plsc.VectorSubcoreMesh(
        core_axis_name="c", subcore_axis_name="s",
        num_cores=N_CORES, num_subcores=N_SUBCORES)
    f = pl.kernel(
        _body,
        out_type=jax.ShapeDtypeStruct((N_SEG, N_FEAT), jnp.float32),
        mesh=mesh,
        compiler_params=pltpu.CompilerParams(use_tc_tiling_on_sc=False),
        scratch_types=[
            pltpu.VMEM((IDS_BUF,), jnp.int32),
            pltpu.VMEM((CHUNK, COLS_PER_C), jnp.float32),
            pltpu.VMEM((CHUNK, COLS_PER_C), jnp.float32),
            pltpu.VMEM((N_SEG, COLS_PER_C), jnp.float32),
            pltpu.VMEM((32, COLS_PER_C), jnp.float32),
            pltpu.VMEM((16, 32), jnp.int32),
            pltpu.SemaphoreType.DMA,
            pltpu.SemaphoreType.DMA,
            pltpu.VMEM_SHARED((N_SEG, COLS_PER_C), jnp.float32),
        ],
    )
    return f(x, ids32)
